# baseline (speedup 1.0000x reference)
import functools

import jax
import jax.numpy as jnp
from jax import lax
from jax.experimental import pallas as pl
from jax.experimental.pallas import tpu as pltpu
from jax.experimental.pallas import tpu_sc as plsc

Z_LOSS_COEF = 0.001
AUX_LOSS_COEF = 0.01

G = 4
T = 8192
E = 64
K = 2

NC = 2
NS = 16
NW = NC * NS
TOK_PER_W = (G * T) // NW
IDX_PER_W = TOK_PER_W * K
ROWS = IDX_PER_W // 128

R = T // 2


def _sc_hist_body(idx_hbm, out_hbm, idx_raw, scat_idx, scat_val, zbuf,
                  hist_sh, sem):
    c = lax.axis_index("c")
    s = lax.axis_index("s")
    wid = c * NS + s
    g = wid // (NW // G)
    off = (wid % (NW // G)) * IDX_PER_W
    pltpu.sync_copy(idx_hbm.at[g, pl.ds(off, IDX_PER_W)], idx_raw)

    gbase = g * E
    lane = lax.iota(jnp.int32, 16)
    odd = (lane % 2) == 1
    perm = lane ^ 1

    def row(j, carry):
        for l in range(8):
            w = idx_raw[pl.ds(j * 128 + l * 16, 16)]
            partner = lax.gather(
                w, perm[:, None],
                lax.GatherDimensionNumbers(
                    offset_dims=(), collapsed_slice_dims=(0,),
                    start_index_map=(0,)),
                slice_sizes=(1,),
                mode=lax.GatherScatterMode.PROMISE_IN_BOUNDS)
            dup = odd & (w == partner)
            scat_idx[j, pl.ds(l * 16, 16)] = w + gbase
            scat_val[j, pl.ds(l * 16, 16)] = jnp.where(dup, 0.0, 1.0)
        return carry

    lax.fori_loop(0, ROWS, row, 0)

    @pl.when(s == 0)
    def _():
        for i in range(G * E // 16):
            zbuf[pl.ds(i * 16, 16)] = jnp.zeros((16,), jnp.float32)
        pltpu.sync_copy(zbuf, hist_sh)

    plsc.subcore_barrier()
    copies = [
        pltpu.async_copy(scat_val.at[j], hist_sh.at[scat_idx.at[j]],
                         sem, add=True)
        for j in range(ROWS)
    ]
    for h in copies:
        h.wait()
    plsc.subcore_barrier()

    @pl.when(s == 0)
    def _():
        pltpu.sync_copy(hist_sh, out_hbm.at[c])


def _sc_hist(idx_2d):
    mesh = plsc.VectorSubcoreMesh(core_axis_name="c", subcore_axis_name="s")
    fn = functools.partial(
        pl.kernel,
        mesh=mesh,
        out_type=jax.ShapeDtypeStruct((NC, G * E), jnp.float32),
        scratch_types=[
            pltpu.VMEM((IDX_PER_W,), jnp.int32),
            pltpu.VMEM((ROWS, 128), jnp.int32),
            pltpu.VMEM((ROWS, 128), jnp.float32),
            pltpu.VMEM((G * E,), jnp.float32),
            pltpu.VMEM_SHARED((G * E,), jnp.float32),
            pltpu.SemaphoreType.DMA,
        ],
    )(_sc_hist_body)
    return fn(idx_2d)


def _tc_body(logits_ref, psum_ref, z_ref, zacc_ref):
    g = pl.program_id(0)

    x2 = logits_ref[0]
    m = jnp.max(x2, axis=1, keepdims=True)
    e = jnp.exp(x2 - m)
    li = lax.broadcasted_iota(jnp.int32, (2 * E, 2 * E), 0)
    lj = lax.broadcasted_iota(jnp.int32, (2 * E, 2 * E), 1)
    seg = ((li // E) == (lj // E)).astype(jnp.float32)
    s128 = lax.dot_general(e, seg, (((1,), (0,)), ((), ())),
                           preferred_element_type=jnp.float32)
    p = e * (1.0 / s128)
    lz = m + jnp.log(s128)
    lz2 = lz * lz
    ones_t = jnp.full((1, R), 1.0, jnp.float32)
    psum128 = lax.dot_general(ones_t, p, (((1,), (0,)), ((), ())),
                              preferred_element_type=jnp.float32)
    zc = lax.dot_general(ones_t, lz2, (((1,), (0,)), ((), ())),
                         preferred_element_type=jnp.float32)
    psum_ref[0] = psum128[:, :E] + psum128[:, E:]

    @pl.when(g == 0)
    def _():
        zacc_ref[0, 0] = 0.0

    zacc_ref[0, 0] += jnp.sum(zc) * (1.0 / E)

    @pl.when(g == G - 1)
    def _():
        z_ref[...] = jnp.full((1, 1), zacc_ref[0, 0], jnp.float32)


def _tc_main(logits2):
    return pl.pallas_call(
        _tc_body,
        grid=(G,),
        in_specs=[pl.BlockSpec((1, R, 2 * E), lambda g: (g, 0, 0))],
        out_specs=[
            pl.BlockSpec((1, 1, E), lambda g: (g, 0, 0)),
            pl.BlockSpec((1, 1), lambda g: (0, 0)),
        ],
        out_shape=[
            jax.ShapeDtypeStruct((G, 1, E), jnp.float32),
            jax.ShapeDtypeStruct((1, 1), jnp.float32),
        ],
        scratch_shapes=[pltpu.SMEM((1, 1), jnp.float32)],
    )(logits2)


def kernel(router_logits, expert_indexes):
    if expert_indexes.dtype != jnp.int32:
        expert_indexes = expert_indexes.astype(jnp.int32)
    idx_2d = jnp.reshape(expert_indexes, (G, T * K))
    logits2 = jnp.reshape(router_logits, (G, R, 2 * E))
    cnt = _sc_hist(idx_2d)
    psum, z = _tc_main(logits2)
    psum = jnp.reshape(psum, (G, E))
    cnt_g = jnp.reshape(cnt, (NC, G, E)).sum(axis=0)
    z_loss = z[0, 0] / (G * T)
    aux_loss = jnp.sum(cnt_g * psum) * E / (T * T * G)
    return Z_LOSS_COEF * z_loss + AUX_LOSS_COEF * aux_loss
